# initial kernel scaffold (unmeasured)
import jax
import jax.numpy as jnp
from jax import lax
from jax.experimental import pallas as pl
from jax.experimental.pallas import tpu as pltpu

N_DEV = 4
SQ = 2048
DM = 1024
HQ = 8
DH = 128
HD = HQ * DH
CHUNK = SQ // N_DEV
SCALE = 0.08838834764831843
NEG = -1e9


def kernel(x, Wq, K_ext, V_ext, Wo):
    my = lax.axis_index("i")
    xb = x[0].astype(jnp.bfloat16)
    wq = lax.dynamic_slice_in_dim(Wq, my * HD, HD, 1).astype(jnp.bfloat16)
    wo = lax.dynamic_slice_in_dim(Wo, my * HD, HD, 0).astype(jnp.bfloat16)
    kb = K_ext[0].astype(jnp.bfloat16)
    vb = V_ext[0].astype(jnp.bfloat16)

    def body(x_ref, wq_ref, k_ref, v_ref, wo_ref, out_ref,
             acc_ref, mask_ref, sbuf_ref, rbuf_ref, send_sems, recv_sems):
        p = lax.axis_index("i")
        left = (p - 1 + N_DEV) % N_DEV
        right = (p + 1) % N_DEV

        barrier_sem = pltpu.get_barrier_semaphore()
        for nbr in (left, right):
            pl.semaphore_signal(barrier_sem, inc=1, device_id=(nbr,),
                                device_id_type=pl.DeviceIdType.MESH)
        pl.semaphore_wait(barrier_sem, 2)

        qblk = lax.broadcasted_iota(jnp.int32, (SQ, SQ), 0) // 64
        kblk = lax.broadcasted_iota(jnp.int32, (SQ, SQ), 1) // 64
        keep = (qblk == kblk) | (kblk == 0) | (((qblk + kblk) % 3) == 0)
        mask_ref[...] = jnp.where(keep, 0.0, NEG).astype(jnp.bfloat16)

        q = jnp.dot(x_ref[...], wq_ref[...],
                    preferred_element_type=jnp.float32).astype(jnp.bfloat16)

        for h in range(HQ):
            qh = q[:, h * DH:(h + 1) * DH]
            kh = k_ref[:, h, :]
            s = lax.dot_general(qh, kh, (((1,), (1,)), ((), ())),
                                preferred_element_type=jnp.float32)
            s = s * SCALE + mask_ref[...].astype(jnp.float32)
            m = jnp.max(s, axis=-1, keepdims=True)
            w = jnp.exp(s - m)
            w = (w / jnp.sum(w, axis=-1, keepdims=True)).astype(jnp.bfloat16)
            ctx = jnp.dot(w, v_ref[:, h, :],
                          preferred_element_type=jnp.float32)
            part = jnp.dot(ctx.astype(jnp.bfloat16), wo_ref[h * DH:(h + 1) * DH, :],
                           preferred_element_type=jnp.float32)
            if h == 0:
                acc_ref[...] = part
            else:
                acc_ref[...] += part

        def rows(i):
            return pl.ds(i * CHUNK, CHUNK)

        for h in range(N_DEV - 1):
            s_idx = (p - h + N_DEV) % N_DEV
            payload = acc_ref[rows(s_idx), :]
            if h > 0:
                payload = payload + rbuf_ref[h - 1].astype(jnp.float32)
            sbuf_ref[h, :, :] = payload.astype(jnp.bfloat16)
            rdma = pltpu.make_async_remote_copy(
                src_ref=sbuf_ref.at[h], dst_ref=rbuf_ref.at[h],
                send_sem=send_sems.at[h], recv_sem=recv_sems.at[h],
                device_id=(right,), device_id_type=pl.DeviceIdType.MESH)
            rdma.start()
            rdma.wait()

        g = (p + 1) % N_DEV
        full = acc_ref[rows(g), :] + rbuf_ref[N_DEV - 2].astype(jnp.float32)
        out_ref[rows(g), :] = full
        sbuf_ref[N_DEV - 1, :, :] = full.astype(jnp.bfloat16)

        for h in range(N_DEV - 1, 2 * (N_DEV - 1)):
            src = sbuf_ref.at[N_DEV - 1] if h == N_DEV - 1 else rbuf_ref.at[h - 1]
            rdma = pltpu.make_async_remote_copy(
                src_ref=src, dst_ref=rbuf_ref.at[h],
                send_sem=send_sems.at[h], recv_sem=recv_sems.at[h],
                device_id=(right,), device_id_type=pl.DeviceIdType.MESH)
            rdma.start()
            rdma.wait()
            o_idx = (p - (h - (N_DEV - 1)) + N_DEV) % N_DEV
            out_ref[rows(o_idx), :] = rbuf_ref[h].astype(jnp.float32)

    out = pl.pallas_call(
        body,
        out_shape=jax.ShapeDtypeStruct((SQ, DM), jnp.float32),
        in_specs=[pl.BlockSpec(memory_space=pltpu.VMEM)] * 5,
        out_specs=pl.BlockSpec(memory_space=pltpu.VMEM),
        scratch_shapes=[
            pltpu.VMEM((SQ, DM), jnp.float32),
            pltpu.VMEM((SQ, SQ), jnp.bfloat16),
            pltpu.VMEM((N_DEV, CHUNK, DM), jnp.bfloat16),
            pltpu.VMEM((2 * (N_DEV - 1), CHUNK, DM), jnp.bfloat16),
            pltpu.SemaphoreType.DMA((2 * (N_DEV - 1),)),
            pltpu.SemaphoreType.DMA((2 * (N_DEV - 1),)),
        ],
        compiler_params=pltpu.CompilerParams(collective_id=0),
    )(xb, wq, kb, vb, wo)
    return out[None]


# baseline (device time: 249590 ns/iter reference)
import jax
import jax.numpy as jnp
from jax import lax
from jax.experimental import pallas as pl
from jax.experimental.pallas import tpu as pltpu

N_DEV = 4
SQ = 2048
DM = 1024
HQ = 8
DH = 128
HD = HQ * DH
CHUNK = SQ // N_DEV
SCALE = 0.08838834764831843
NEG = -1e9


def kernel(x, Wq, K_ext, V_ext, Wo):
    my = lax.axis_index("i")
    xb = x[0].astype(jnp.bfloat16)
    wq = lax.dynamic_slice_in_dim(Wq, my * HD, HD, 1).astype(jnp.bfloat16)
    wo = lax.dynamic_slice_in_dim(Wo, my * HD, HD, 0).astype(jnp.bfloat16)
    wq3 = wq.reshape(DM, HQ, DH).transpose(1, 0, 2)
    wo3 = wo.reshape(HQ, DH, DM)
    k3 = K_ext[0].transpose(1, 0, 2).astype(jnp.bfloat16)
    v3 = V_ext[0].transpose(1, 0, 2).astype(jnp.bfloat16)

    def body(x_ref, wq_ref, k_ref, v_ref, wo_ref, out_ref,
             acc_ref, mask_ref, sbuf_ref, rbuf_ref, send_sems, recv_sems):
        p = lax.axis_index("i")
        left = (p - 1 + N_DEV) % N_DEV
        right = (p + 1) % N_DEV

        barrier_sem = pltpu.get_barrier_semaphore()
        for nbr in (left, right):
            pl.semaphore_signal(barrier_sem, inc=1, device_id=(nbr,),
                                device_id_type=pl.DeviceIdType.MESH)
        pl.semaphore_wait(barrier_sem, 2)

        def rows(i):
            return pl.ds(i * CHUNK, CHUNK)

        acc_ref[...] = jnp.zeros((SQ, DM), jnp.float32)

        def t_loop(t, c):
            qblk = (lax.broadcasted_iota(jnp.int32, (CHUNK, SQ), 0) + t * CHUNK) // 64
            kblk = lax.broadcasted_iota(jnp.int32, (CHUNK, SQ), 1) // 64
            keep = (qblk == kblk) | (kblk == 0) | (((qblk + kblk) % 3) == 0)
            mask_ref[...] = jnp.where(keep, 0.0, NEG).astype(jnp.bfloat16)

            def h_loop(h, c2):
                qh = jnp.dot(x_ref[rows(t), :], wq_ref[h, :, :],
                             preferred_element_type=jnp.float32
                             ).astype(jnp.bfloat16)
                s = lax.dot_general(qh, k_ref[h, :, :],
                                    (((1,), (1,)), ((), ())),
                                    preferred_element_type=jnp.float32)
                s = s * SCALE + mask_ref[...].astype(jnp.float32)
                m = jnp.max(s, axis=-1, keepdims=True)
                w = jnp.exp(s - m)
                w = (w / jnp.sum(w, axis=-1, keepdims=True)).astype(jnp.bfloat16)
                ctx = jnp.dot(w, v_ref[h, :, :],
                              preferred_element_type=jnp.float32)
                part = jnp.dot(ctx.astype(jnp.bfloat16), wo_ref[h, :, :],
                               preferred_element_type=jnp.float32)
                acc_ref[rows(t), :] += part
                return c2
            return lax.fori_loop(0, HQ, h_loop, c)
        lax.fori_loop(0, N_DEV, t_loop, 0)

        for h in range(N_DEV - 1):
            s_idx = (p - h + N_DEV) % N_DEV
            payload = acc_ref[rows(s_idx), :]
            if h > 0:
                payload = payload + rbuf_ref[h - 1].astype(jnp.float32)
            sbuf_ref[h, :, :] = payload.astype(jnp.bfloat16)
            rdma = pltpu.make_async_remote_copy(
                src_ref=sbuf_ref.at[h], dst_ref=rbuf_ref.at[h],
                send_sem=send_sems.at[h], recv_sem=recv_sems.at[h],
                device_id=(right,), device_id_type=pl.DeviceIdType.MESH)
            rdma.start()
            rdma.wait()

        g = (p + 1) % N_DEV
        full = acc_ref[rows(g), :] + rbuf_ref[N_DEV - 2].astype(jnp.float32)
        out_ref[rows(g), :] = full
        sbuf_ref[N_DEV - 1, :, :] = full.astype(jnp.bfloat16)

        for h in range(N_DEV - 1, 2 * (N_DEV - 1)):
            src = sbuf_ref.at[N_DEV - 1] if h == N_DEV - 1 else rbuf_ref.at[h - 1]
            rdma = pltpu.make_async_remote_copy(
                src_ref=src, dst_ref=rbuf_ref.at[h],
                send_sem=send_sems.at[h], recv_sem=recv_sems.at[h],
                device_id=(right,), device_id_type=pl.DeviceIdType.MESH)
            rdma.start()
            rdma.wait()
            o_idx = (p - (h - (N_DEV - 1)) + N_DEV) % N_DEV
            out_ref[rows(o_idx), :] = rbuf_ref[h].astype(jnp.float32)

    out = pl.pallas_call(
        body,
        out_shape=jax.ShapeDtypeStruct((SQ, DM), jnp.float32),
        in_specs=[pl.BlockSpec(memory_space=pltpu.VMEM)] * 5,
        out_specs=pl.BlockSpec(memory_space=pltpu.VMEM),
        scratch_shapes=[
            pltpu.VMEM((SQ, DM), jnp.float32),
            pltpu.VMEM((CHUNK, SQ), jnp.bfloat16),
            pltpu.VMEM((N_DEV, CHUNK, DM), jnp.bfloat16),
            pltpu.VMEM((2 * (N_DEV - 1), CHUNK, DM), jnp.bfloat16),
            pltpu.SemaphoreType.DMA((2 * (N_DEV - 1),)),
            pltpu.SemaphoreType.DMA((2 * (N_DEV - 1),)),
        ],
        compiler_params=pltpu.CompilerParams(
            collective_id=0, vmem_limit_bytes=100 * 1024 * 1024),
    )(xb, wq3, k3, v3, wo3)
    return out[None]


# device time: 179629 ns/iter; 1.3895x vs baseline; 1.3895x over previous
import jax
import jax.numpy as jnp
from jax import lax
from jax.experimental import pallas as pl
from jax.experimental.pallas import tpu as pltpu

N_DEV = 4
SQ = 2048
DM = 1024
HQ = 8
DH = 128
HD = HQ * DH
CHUNK = SQ // N_DEV
SCALE = 0.08838834764831843
NEG = -1e9


def kernel(x, Wq, K_ext, V_ext, Wo):
    my = lax.axis_index("i")
    xb = x[0].astype(jnp.bfloat16)
    wq = (lax.dynamic_slice_in_dim(Wq, my * HD, HD, 1) * SCALE).astype(jnp.bfloat16)
    wo = lax.dynamic_slice_in_dim(Wo, my * HD, HD, 0).astype(jnp.bfloat16)
    wq3 = wq.reshape(DM, HQ, DH).transpose(1, 0, 2)
    wo3 = wo.reshape(HQ, DH, DM)
    k3 = K_ext[0].transpose(1, 0, 2).astype(jnp.bfloat16)
    v3 = V_ext[0].transpose(1, 0, 2).astype(jnp.bfloat16)

    def body(x_ref, wq_ref, k_ref, v_ref, wo_ref, out_ref,
             mask_ref, sbuf_ref, rbuf_ref, send_sems, recv_sems):
        p = lax.axis_index("i")
        left = (p - 1 + N_DEV) % N_DEV
        right = (p + 1) % N_DEV

        barrier_sem = pltpu.get_barrier_semaphore()
        for nbr in (left, right):
            pl.semaphore_signal(barrier_sem, inc=1, device_id=(nbr,),
                                device_id_type=pl.DeviceIdType.MESH)
        pl.semaphore_wait(barrier_sem, 2)

        def rows(i):
            return pl.ds(i * CHUNK, CHUNK)

        def ring_rdma(src, dst_slot, sem):
            return pltpu.make_async_remote_copy(
                src_ref=src, dst_ref=rbuf_ref.at[dst_slot],
                send_sem=send_sems.at[sem], recv_sem=recv_sems.at[sem],
                device_id=(right,), device_id_type=pl.DeviceIdType.MESH)

        def attn_tile(t_idx):
            qblk = (lax.broadcasted_iota(jnp.int32, (CHUNK, SQ), 0)
                    + t_idx * CHUNK) // 64
            kblk = lax.broadcasted_iota(jnp.int32, (CHUNK, SQ), 1) // 64
            keep = (qblk == kblk) | (kblk == 0) | (((qblk + kblk) % 3) == 0)
            mask_ref[...] = jnp.where(keep, 0.0, NEG)

            def h_loop(h, acc_t):
                qh = jnp.dot(x_ref[rows(t_idx), :], wq_ref[h, :, :],
                             preferred_element_type=jnp.float32
                             ).astype(jnp.bfloat16)
                s = lax.dot_general(qh, k_ref[h, :, :],
                                    (((1,), (1,)), ((), ())),
                                    preferred_element_type=jnp.float32)
                w = jnp.exp(s + mask_ref[...])
                denom = jnp.sum(w, axis=-1, keepdims=True)
                ctx = jnp.dot(w.astype(jnp.bfloat16), v_ref[h, :, :],
                              preferred_element_type=jnp.float32)
                ctx = (ctx / denom).astype(jnp.bfloat16)
                return acc_t + jnp.dot(ctx, wo_ref[h, :, :],
                                       preferred_element_type=jnp.float32)

            return lax.fori_loop(
                0, HQ, h_loop, jnp.zeros((CHUNK, DM), jnp.float32))

        rs = [None] * (N_DEV - 1)
        for h in range(N_DEV):
            t_idx = (p - h + N_DEV) % N_DEV
            payload = attn_tile(t_idx)
            if h > 0:
                rs[h - 1].wait()
                payload = payload + rbuf_ref[h - 1].astype(jnp.float32)
            if h < N_DEV - 1:
                sbuf_ref[h, :, :] = payload.astype(jnp.bfloat16)
                rs[h] = ring_rdma(sbuf_ref.at[h], h, h)
                rs[h].start()
            else:
                out_ref[rows(t_idx), :] = payload
                sbuf_ref[N_DEV - 1, :, :] = payload.astype(jnp.bfloat16)

        for h in range(N_DEV - 1, 2 * (N_DEV - 1)):
            src = sbuf_ref.at[N_DEV - 1] if h == N_DEV - 1 else rbuf_ref.at[h - 1]
            rdma = ring_rdma(src, h, h)
            rdma.start()
            rdma.wait()
            o_idx = (p - (h - (N_DEV - 1)) + N_DEV) % N_DEV
            out_ref[rows(o_idx), :] = rbuf_ref[h].astype(jnp.float32)

    out = pl.pallas_call(
        body,
        out_shape=jax.ShapeDtypeStruct((SQ, DM), jnp.float32),
        in_specs=[pl.BlockSpec(memory_space=pltpu.VMEM)] * 5,
        out_specs=pl.BlockSpec(memory_space=pltpu.VMEM),
        scratch_shapes=[
            pltpu.VMEM((CHUNK, SQ), jnp.float32),
            pltpu.VMEM((N_DEV, CHUNK, DM), jnp.bfloat16),
            pltpu.VMEM((2 * (N_DEV - 1), CHUNK, DM), jnp.bfloat16),
            pltpu.SemaphoreType.DMA((2 * (N_DEV - 1),)),
            pltpu.SemaphoreType.DMA((2 * (N_DEV - 1),)),
        ],
        compiler_params=pltpu.CompilerParams(
            collective_id=0, vmem_limit_bytes=100 * 1024 * 1024),
    )(xb, wq3, k3, v3, wo3)
    return out[None]
